# baseline (speedup 1.0000x reference)
import functools

import jax
import jax.numpy as jnp
from jax import lax
from jax.experimental import pallas as pl
from jax.experimental.pallas import tpu as pltpu
from jax.experimental.pallas import tpu_sc as plsc

_B, _T, _A, _F = 2, 4, 2048, 128
_E = 32768
_DE = 16
_R = 128
_NRES = 128
_BT = _B * _T

_NPART = 4
_EPT = _E // _NPART
_VPT = _EPT // 16
_CROWS = _NRES // _NPART
_CCELLS = _CROWS * _NRES


def _sc_kernel_body(srcdst, ca, resmap, enc_out, idx_out,
                    src_v, dst_v, ca_v, res_v, grid_v, bit_v,
                    enc_a, enc_b, idx_row, shared):
    c = lax.axis_index("c")
    s = lax.axis_index("s")
    sl = c * 4 + s // 4
    ls = s // 4
    k = s % 4
    b = sl // _T

    lane = lax.broadcasted_iota(jnp.int32, (16,), 0)
    one16 = jnp.full((16,), 1, jnp.int32)

    pltpu.sync_copy(resmap.at[pl.ds(b * _A, _A)], res_v)
    pltpu.sync_copy(ca.at[pl.ds(sl * _A, _A)], ca_v)
    ebase = sl * (2 * _E)
    pltpu.sync_copy(srcdst.at[pl.ds(ebase + k * _EPT, _EPT)], src_v)
    pltpu.sync_copy(srcdst.at[pl.ds(ebase + _E + k * _EPT, _EPT)], dst_v)

    def init_body(i, _):
        grid_v[pl.ds(i * 16, 16)] = jnp.full((16,), -1, jnp.int32)
        bit_v[pl.ds(i * 16, 16)] = jnp.zeros((16,), jnp.int32)
        return 0
    lax.fori_loop(0, (_NRES * _NRES) // 16, init_body, 0, unroll=4)

    def scatter_pass(phase):
        enc_base = k * _EPT + (phase * _E)

        def body(i, _):
            sv = src_v[pl.ds(i * 16, 16)]
            dv = dst_v[pl.ds(i * 16, 16)]
            cs = plsc.load_gather(ca_v, [sv])
            cd = plsc.load_gather(ca_v, [dv])
            keep = (cs > 0.5) & (cd > 0.5)
            rs = plsc.load_gather(res_v, [sv])
            rd = plsc.load_gather(res_v, [dv])
            if phase == 0:
                cells = rs * _NRES + rd
            else:
                cells = rd * _NRES + rs
            enc = (enc_base + i * 16) + lane
            plsc.addupdate_scatter(bit_v, [cells],
                                   lax.shift_left(one16, lane), mask=keep)
            g = plsc.load_gather(bit_v, [cells])
            above = -lax.shift_left(one16 + one16, lane)
            keep_w = keep & ((g & above) == 0)
            plsc.store_scatter(bit_v, [cells], jnp.zeros((16,), jnp.int32),
                               mask=keep)
            plsc.store_scatter(grid_v, [cells], enc, mask=keep_w)
            return 0
        lax.fori_loop(0, _VPT, body, 0)

    scatter_pass(0)
    scatter_pass(1)

    pltpu.sync_copy(grid_v, shared.at[ls, k])
    plsc.subcore_barrier()

    off = k * _CCELLS
    pltpu.sync_copy(shared.at[ls, 0, pl.ds(off, _CCELLS)], enc_a)

    def merge_part(p):
        pltpu.sync_copy(shared.at[ls, p, pl.ds(off, _CCELLS)], enc_b)

        def mbody(j, _):
            va = enc_a[pl.ds(j * 16, 16)]
            vb = enc_b[pl.ds(j * 16, 16)]
            enc_a[pl.ds(j * 16, 16)] = jnp.maximum(va, vb)
            return 0
        lax.fori_loop(0, _CCELLS // 16, mbody, 0, unroll=4)

    merge_part(1)
    merge_part(2)
    merge_part(3)

    out_base = sl * (_NRES * _NRES) + off
    pltpu.sync_copy(enc_a, enc_out.at[pl.ds(out_base, _CCELLS)])

    def idx_body(jj, _):
        v = enc_a[pl.ds(jj * 16, 16)]
        has = v >= 0
        e = jnp.where(v >= _E, v - _E, v)
        dummy = (off + jj * 16) + lane
        e = jnp.where(has, e, dummy)
        idx_row[pl.ds(jj * 16, 16)] = e + sl * _E
        return 0
    lax.fori_loop(0, _CCELLS // 16, idx_body, 0, unroll=4)
    pltpu.sync_copy(idx_row, idx_out.at[pl.ds(out_base, _CCELLS)])


def _scg_kernel_body(idx_in, ea, easel, idx_v, attr_v, sem):
    c = lax.axis_index("c")
    s = lax.axis_index("s")
    sl = c * 4 + s // 4
    k = s % 4
    out_base = sl * (_NRES * _NRES) + k * _CCELLS
    half = _CCELLS // 2
    pltpu.sync_copy(idx_in.at[pl.ds(out_base, _CCELLS)], idx_v)
    cp = pltpu.async_copy(ea.at[idx_v.at[pl.ds(0, half)]], attr_v, sem)
    cp.wait()
    pltpu.sync_copy(attr_v, easel.at[pl.ds(out_base, half)])
    cp = pltpu.async_copy(ea.at[idx_v.at[pl.ds(half, half)]], attr_v, sem)
    cp.wait()
    pltpu.sync_copy(attr_v, easel.at[pl.ds(out_base + half, half)])


def _sc_call(srcdst, ca, resmap):
    kern = pl.kernel(
        _sc_kernel_body,
        out_type=(
            jax.ShapeDtypeStruct((_BT * _NRES * _NRES,), jnp.int32),
            jax.ShapeDtypeStruct((_BT * _NRES * _NRES,), jnp.int32),
        ),
        mesh=plsc.VectorSubcoreMesh(core_axis_name="c", subcore_axis_name="s"),
        compiler_params=pltpu.CompilerParams(needs_layout_passes=False,
                                             use_tc_tiling_on_sc=False),
        scratch_types=[
            pltpu.VMEM((_EPT,), jnp.int32),
            pltpu.VMEM((_EPT,), jnp.int32),
            pltpu.VMEM((_A,), jnp.float32),
            pltpu.VMEM((_A,), jnp.int32),
            pltpu.VMEM((_NRES * _NRES,), jnp.int32),
            pltpu.VMEM((_NRES * _NRES,), jnp.int32),
            pltpu.VMEM((_CCELLS,), jnp.int32),
            pltpu.VMEM((_CCELLS,), jnp.int32),
            pltpu.VMEM((_CCELLS,), jnp.int32),
            pltpu.VMEM_SHARED((4, _NPART, _NRES * _NRES), jnp.int32),
        ],
    )
    return kern(srcdst, ca, resmap)


def _scg_call(idx_flat, ea):
    kern = pl.kernel(
        _scg_kernel_body,
        out_type=jax.ShapeDtypeStruct((_BT * _NRES * _NRES, _DE),
                                      jnp.float32),
        mesh=plsc.VectorSubcoreMesh(core_axis_name="c", subcore_axis_name="s"),
        compiler_params=pltpu.CompilerParams(needs_layout_passes=False,
                                             use_tc_tiling_on_sc=False),
        scratch_types=[
            pltpu.VMEM((_CCELLS,), jnp.int32),
            pltpu.VMEM((_CCELLS // 2, _DE), jnp.float32),
            pltpu.SemaphoreType.DMA,
        ],
    )
    return kern(idx_flat, ea)


def _tca_kernel_body(feats_ref, ressub_ref, reslane_ref,
                     Wa_ref, ba_ref, Wc_ref,
                     node_out_ref, p1_ref, p2_ref):
    feats = feats_ref[0, 0]
    ids = ressub_ref[0]
    x = feats
    s = 1
    while s < _A:
        xs = jnp.concatenate(
            [jnp.full((s, _F), -jnp.inf, jnp.float32), x[:-s]], axis=0)
        ids_s = jnp.concatenate(
            [jnp.full((s, _F), -1, jnp.int32), ids[:-s]], axis=0)
        x = jnp.where(ids_s == ids, jnp.maximum(x, xs), x)
        s *= 2
    res_row = reslane_ref[0, 0:1, :]
    r_col = lax.broadcasted_iota(jnp.int32, (_NRES, _A), 0)
    le = (res_row <= r_col).astype(jnp.float32)
    hi = jnp.sum(le, axis=1, keepdims=True)
    cnt = jnp.sum((res_row == r_col).astype(jnp.float32),
                  axis=1, keepdims=True)
    occ = cnt > 0.5
    a_io = lax.broadcasted_iota(jnp.int32, (_NRES, _A), 1)
    sel = (a_io == (hi.astype(jnp.int32) - 1)).astype(jnp.float32)
    segmax = jnp.dot(sel, x, preferred_element_type=jnp.float32)
    aggregated = jnp.where(occ, segmax, 0.0)
    proj = jnp.dot(aggregated, Wa_ref[...],
                   preferred_element_type=jnp.float32) + ba_ref[...]
    node_out_ref[0, 0] = jnp.where(occ, proj, 0.0)
    p1_ref[0, 0] = jnp.dot(proj, Wc_ref[0:_R],
                           preferred_element_type=jnp.float32)
    p2_ref[0, 0] = jnp.dot(proj, Wc_ref[_R:2 * _R],
                           preferred_element_type=jnp.float32)


def _tca_call(feats, ressub, reslane, Wa, ba2, Wc):
    grid = (_B, _T)
    out_shapes = (
        jax.ShapeDtypeStruct((_B, _T, _NRES, _R), jnp.float32),
        jax.ShapeDtypeStruct((_B, _T, _NRES, _R), jnp.float32),
        jax.ShapeDtypeStruct((_B, _T, _NRES, _R), jnp.float32),
    )
    return pl.pallas_call(
        _tca_kernel_body,
        grid=grid,
        in_specs=[
            pl.BlockSpec((1, 1, _A, _F), lambda b, t: (b, t, 0, 0)),
            pl.BlockSpec((1, _A, _F), lambda b, t: (b, 0, 0)),
            pl.BlockSpec((1, 8, _A), lambda b, t: (b, 0, 0)),
            pl.BlockSpec((_F, _R), lambda b, t: (0, 0)),
            pl.BlockSpec((1, _R), lambda b, t: (0, 0)),
            pl.BlockSpec((3 * _R, _R), lambda b, t: (0, 0)),
        ],
        out_specs=(
            pl.BlockSpec((1, 1, _NRES, _R), lambda b, t: (b, t, 0, 0)),
            pl.BlockSpec((1, 1, _NRES, _R), lambda b, t: (b, t, 0, 0)),
            pl.BlockSpec((1, 1, _NRES, _R), lambda b, t: (b, t, 0, 0)),
        ),
        out_shape=out_shapes,
    )(feats, ressub, reslane, Wa, ba2, Wc)


def _tcb_kernel_body(enc_ref, attr_ref, p1_ref, p2_ref,
                     We_ref, be_ref, Wc_ref, bc_ref, gam_ref, bet_ref,
                     edge_out_ref, eaw_ref):
    attr = attr_ref[0, 0]
    Wc3 = Wc_ref[2 * _R:3 * _R]
    Wep = jnp.dot(We_ref[...], Wc3, preferred_element_type=jnp.float32)
    bprime = bc_ref[...] + jnp.dot(be_ref[...], Wc3,
                                   preferred_element_type=jnp.float32)
    ri = lax.broadcasted_iota(jnp.int32, (_R, 8 * _R), 0)
    ci = lax.broadcasted_iota(jnp.int32, (_R, 8 * _R), 1)
    wtile = jnp.tile(Wep, (8, 8))
    wbig = jnp.where((ri // _DE) == (ci // _R), wtile, 0.0)
    eaw = jnp.dot(attr, wbig, preferred_element_type=jnp.float32)
    eaw_ref[...] = (
        eaw.reshape(_NRES * _NRES // 8, 8, _R).reshape(_NRES * _NRES, _R))
    P1 = p1_ref[0, 0]
    P2 = p2_ref[0, 0]
    enc2 = enc_ref[0, 0]
    phf = (enc2 < _E).astype(jnp.float32)
    hasf = (enc2 >= 0).astype(jnp.float32)
    gam = gam_ref[...]
    bet = bet_ref[...]
    j_io = lax.broadcasted_iota(jnp.int32, (_NRES, 1), 0)

    def row_body(i, _):
        oh = (j_io == i).astype(jnp.float32)
        ph = lax.dot_general(phf, oh, (((0,), (0,)), ((), ())),
                             preferred_element_type=jnp.float32) > 0.5
        has = lax.dot_general(hasf, oh, (((0,), (0,)), ((), ())),
                              preferred_element_type=jnp.float32) > 0.5
        p1row = p1_ref[0, 0, pl.ds(i, 1), :]
        p2row = p2_ref[0, 0, pl.ds(i, 1), :]
        a1 = p1row + P2
        a2 = p2row + P1
        eaw_i = eaw_ref[pl.ds(i * _NRES, _NRES), :]
        pre = jnp.where(ph, a1, a2) + eaw_i + bprime
        h = jnp.maximum(pre, 0.0)
        mu = jnp.mean(h, axis=-1, keepdims=True)
        var = jnp.mean((h - mu) ** 2, axis=-1, keepdims=True)
        ln = gam * (h - mu) / jnp.sqrt(var + 1e-5) + bet
        out = jnp.where(has, ln, 0.0)
        edge_out_ref[0, 0, pl.ds(i, 1)] = out.reshape(1, _NRES, _R)
        return 0

    lax.fori_loop(0, _NRES, row_body, 0)


def _tcb_call(enc4, attr4, p1, p2, We, be2, Wc, bc2, gam2, bet2):
    grid = (_B, _T)
    return pl.pallas_call(
        _tcb_kernel_body,
        grid=grid,
        in_specs=[
            pl.BlockSpec((1, 1, _NRES, _NRES), lambda b, t: (b, t, 0, 0)),
            pl.BlockSpec((1, 1, _NRES * _NRES // 8, _R),
                         lambda b, t: (b, t, 0, 0)),
            pl.BlockSpec((1, 1, _NRES, _R), lambda b, t: (b, t, 0, 0)),
            pl.BlockSpec((1, 1, _NRES, _R), lambda b, t: (b, t, 0, 0)),
            pl.BlockSpec((_DE, _R), lambda b, t: (0, 0)),
            pl.BlockSpec((1, _R), lambda b, t: (0, 0)),
            pl.BlockSpec((3 * _R, _R), lambda b, t: (0, 0)),
            pl.BlockSpec((1, _R), lambda b, t: (0, 0)),
            pl.BlockSpec((1, _R), lambda b, t: (0, 0)),
            pl.BlockSpec((1, _R), lambda b, t: (0, 0)),
        ],
        out_specs=pl.BlockSpec((1, 1, _NRES, _NRES, _R),
                               lambda b, t: (b, t, 0, 0, 0)),
        out_shape=jax.ShapeDtypeStruct((_B, _T, _NRES, _NRES, _R),
                                       jnp.float32),
        scratch_shapes=[pltpu.VMEM((_NRES * _NRES, _R), jnp.float32)],
    )(enc4, attr4, p1, p2, We, be2, Wc, bc2, gam2, bet2)


def kernel(atom_features, residue_indices, edge_index, edge_attr,
           Wa, ba, We, be, Wc, bc, gamma, beta):
    atom_features = atom_features.astype(jnp.float32)
    residue_indices = residue_indices.astype(jnp.int32)
    edge_index = edge_index.astype(jnp.int32)
    edge_attr = edge_attr.astype(jnp.float32)

    srcdst = edge_index.reshape(-1)
    ca = atom_features[..., 1].reshape(-1)
    resflat = residue_indices.reshape(-1)
    eaflat = edge_attr.reshape(-1, _DE)

    enc_flat, idx_flat = _sc_call(srcdst, ca, resflat)
    easel = _scg_call(idx_flat, eaflat)

    enc4 = enc_flat.reshape(_B, _T, _NRES, _NRES)
    attr4 = easel.reshape(_B, _T, _NRES * _NRES // 8, _R)
    ressub = jnp.broadcast_to(residue_indices[:, :, None], (_B, _A, _F))
    reslane = jnp.broadcast_to(residue_indices[:, None, :], (_B, 8, _A))

    node_out, p1, p2 = _tca_call(
        atom_features, ressub, reslane, Wa, ba.reshape(1, _R), Wc)
    edge_out = _tcb_call(
        enc4, attr4, p1, p2, We, be.reshape(1, _R), Wc,
        bc.reshape(1, _R), gamma.reshape(1, _R), beta.reshape(1, _R))
    return (edge_out, node_out)


# speedup vs baseline: 2.6043x; 2.6043x over previous
import functools

import jax
import jax.numpy as jnp
from jax import lax
from jax.experimental import pallas as pl
from jax.experimental.pallas import tpu as pltpu
from jax.experimental.pallas import tpu_sc as plsc

_B, _T, _A, _F = 2, 4, 2048, 128
_E = 32768
_DE = 16
_R = 128
_NRES = 128
_BT = _B * _T

_NPART = 4
_EPT = _E // _NPART
_VPT = _EPT // 16
_CROWS = _NRES // _NPART
_CCELLS = _CROWS * _NRES


def _sc_kernel_body(srcdst, ca, resmap, enc_out, idx_out,
                    src_v, dst_v, ca_v, res_v, grid_v, bit_v,
                    enc_a, enc_b, idx_row, shared):
    c = lax.axis_index("c")
    s = lax.axis_index("s")
    sl = c * 4 + s // 4
    ls = s // 4
    k = s % 4
    b = sl // _T

    lane = lax.broadcasted_iota(jnp.int32, (16,), 0)
    one16 = jnp.full((16,), 1, jnp.int32)

    pltpu.sync_copy(resmap.at[pl.ds(b * _A, _A)], res_v)
    pltpu.sync_copy(ca.at[pl.ds(sl * _A, _A)], ca_v)
    ebase = sl * (2 * _E)
    pltpu.sync_copy(srcdst.at[pl.ds(ebase + k * _EPT, _EPT)], src_v)
    pltpu.sync_copy(srcdst.at[pl.ds(ebase + _E + k * _EPT, _EPT)], dst_v)

    def init_body(i, _):
        grid_v[pl.ds(i * 16, 16)] = jnp.full((16,), -1, jnp.int32)
        bit_v[pl.ds(i * 16, 16)] = jnp.zeros((16,), jnp.int32)
        return 0
    lax.fori_loop(0, (_NRES * _NRES) // 16, init_body, 0, unroll=4)

    def scatter_pass(phase):
        enc_base = k * _EPT + (phase * _E)

        def body(i, _):
            sv = src_v[pl.ds(i * 16, 16)]
            dv = dst_v[pl.ds(i * 16, 16)]
            cs = plsc.load_gather(ca_v, [sv])
            cd = plsc.load_gather(ca_v, [dv])
            keep = (cs > 0.5) & (cd > 0.5)
            rs = plsc.load_gather(res_v, [sv])
            rd = plsc.load_gather(res_v, [dv])
            if phase == 0:
                cells = rs * _NRES + rd
            else:
                cells = rd * _NRES + rs
            enc = (enc_base + i * 16) + lane
            plsc.addupdate_scatter(bit_v, [cells],
                                   lax.shift_left(one16, lane), mask=keep)
            g = plsc.load_gather(bit_v, [cells])
            above = -lax.shift_left(one16 + one16, lane)
            keep_w = keep & ((g & above) == 0)
            plsc.store_scatter(bit_v, [cells], jnp.zeros((16,), jnp.int32),
                               mask=keep)
            plsc.store_scatter(grid_v, [cells], enc, mask=keep_w)
            return 0
        lax.fori_loop(0, _VPT, body, 0)

    scatter_pass(0)
    scatter_pass(1)

    pltpu.sync_copy(grid_v, shared.at[ls, k])
    plsc.subcore_barrier()

    off = k * _CCELLS
    pltpu.sync_copy(shared.at[ls, 0, pl.ds(off, _CCELLS)], enc_a)

    def merge_part(p):
        pltpu.sync_copy(shared.at[ls, p, pl.ds(off, _CCELLS)], enc_b)

        def mbody(j, _):
            va = enc_a[pl.ds(j * 16, 16)]
            vb = enc_b[pl.ds(j * 16, 16)]
            enc_a[pl.ds(j * 16, 16)] = jnp.maximum(va, vb)
            return 0
        lax.fori_loop(0, _CCELLS // 16, mbody, 0, unroll=4)

    merge_part(1)
    merge_part(2)
    merge_part(3)

    out_base = sl * (_NRES * _NRES) + off
    pltpu.sync_copy(enc_a, enc_out.at[pl.ds(out_base, _CCELLS)])

    def idx_body(jj, _):
        v = enc_a[pl.ds(jj * 16, 16)]
        has = v >= 0
        e = jnp.where(v >= _E, v - _E, v)
        dummy = (off + jj * 16) + lane
        e = jnp.where(has, e, dummy)
        idx_row[pl.ds(jj * 16, 16)] = e + sl * _E
        return 0
    lax.fori_loop(0, _CCELLS // 16, idx_body, 0, unroll=4)
    pltpu.sync_copy(idx_row, idx_out.at[pl.ds(out_base, _CCELLS)])


def _scg_kernel_body(idx_in, ea, easel, idx_v, attr_v, sem):
    c = lax.axis_index("c")
    s = lax.axis_index("s")
    sl = c * 4 + s // 4
    k = s % 4
    out_base = sl * (_NRES * _NRES) + k * _CCELLS
    half = _CCELLS // 2
    pltpu.sync_copy(idx_in.at[pl.ds(out_base, _CCELLS)], idx_v)
    cp = pltpu.async_copy(ea.at[idx_v.at[pl.ds(0, half)]], attr_v, sem)
    cp.wait()
    pltpu.sync_copy(attr_v, easel.at[pl.ds(out_base, half)])
    cp = pltpu.async_copy(ea.at[idx_v.at[pl.ds(half, half)]], attr_v, sem)
    cp.wait()
    pltpu.sync_copy(attr_v, easel.at[pl.ds(out_base + half, half)])


def _sc_call(srcdst, ca, resmap):
    kern = pl.kernel(
        _sc_kernel_body,
        out_type=(
            jax.ShapeDtypeStruct((_BT * _NRES * _NRES,), jnp.int32),
            jax.ShapeDtypeStruct((_BT * _NRES * _NRES,), jnp.int32),
        ),
        mesh=plsc.VectorSubcoreMesh(core_axis_name="c", subcore_axis_name="s"),
        compiler_params=pltpu.CompilerParams(needs_layout_passes=False,
                                             use_tc_tiling_on_sc=False),
        scratch_types=[
            pltpu.VMEM((_EPT,), jnp.int32),
            pltpu.VMEM((_EPT,), jnp.int32),
            pltpu.VMEM((_A,), jnp.float32),
            pltpu.VMEM((_A,), jnp.int32),
            pltpu.VMEM((_NRES * _NRES,), jnp.int32),
            pltpu.VMEM((_NRES * _NRES,), jnp.int32),
            pltpu.VMEM((_CCELLS,), jnp.int32),
            pltpu.VMEM((_CCELLS,), jnp.int32),
            pltpu.VMEM((_CCELLS,), jnp.int32),
            pltpu.VMEM_SHARED((4, _NPART, _NRES * _NRES), jnp.int32),
        ],
    )
    return kern(srcdst, ca, resmap)


def _scg_call(idx_flat, ea):
    kern = pl.kernel(
        _scg_kernel_body,
        out_type=jax.ShapeDtypeStruct((_BT * _NRES * _NRES, _DE),
                                      jnp.float32),
        mesh=plsc.VectorSubcoreMesh(core_axis_name="c", subcore_axis_name="s"),
        compiler_params=pltpu.CompilerParams(needs_layout_passes=False,
                                             use_tc_tiling_on_sc=False),
        scratch_types=[
            pltpu.VMEM((_CCELLS,), jnp.int32),
            pltpu.VMEM((_CCELLS // 2, _DE), jnp.float32),
            pltpu.SemaphoreType.DMA,
        ],
    )
    return kern(idx_flat, ea)


def _tca_kernel_body(feats_ref, ressub_ref, reslane_ref,
                     Wa_ref, ba_ref, Wc_ref,
                     node_out_ref, p1_ref, p2_ref):
    feats = feats_ref[0, 0]
    ids = ressub_ref[0]
    x = feats
    s = 1
    while s < _A:
        xs = jnp.concatenate(
            [jnp.full((s, _F), -jnp.inf, jnp.float32), x[:-s]], axis=0)
        ids_s = jnp.concatenate(
            [jnp.full((s, _F), -1, jnp.int32), ids[:-s]], axis=0)
        x = jnp.where(ids_s == ids, jnp.maximum(x, xs), x)
        s *= 2
    res_row = reslane_ref[0, 0:1, :]
    r_col = lax.broadcasted_iota(jnp.int32, (_NRES, _A), 0)
    le = (res_row <= r_col).astype(jnp.float32)
    hi = jnp.sum(le, axis=1, keepdims=True)
    cnt = jnp.sum((res_row == r_col).astype(jnp.float32),
                  axis=1, keepdims=True)
    occ = cnt > 0.5
    a_io = lax.broadcasted_iota(jnp.int32, (_NRES, _A), 1)
    sel = (a_io == (hi.astype(jnp.int32) - 1)).astype(jnp.float32)
    segmax = jnp.dot(sel, x, preferred_element_type=jnp.float32)
    aggregated = jnp.where(occ, segmax, 0.0)
    proj = jnp.dot(aggregated, Wa_ref[...],
                   preferred_element_type=jnp.float32) + ba_ref[...]
    node_out_ref[0, 0] = jnp.where(occ, proj, 0.0)
    p1_ref[0, 0] = jnp.dot(proj, Wc_ref[0:_R],
                           preferred_element_type=jnp.float32)
    p2_ref[0, 0] = jnp.dot(proj, Wc_ref[_R:2 * _R],
                           preferred_element_type=jnp.float32)


def _tca_call(feats, ressub, reslane, Wa, ba2, Wc):
    grid = (_B, _T)
    out_shapes = (
        jax.ShapeDtypeStruct((_B, _T, _NRES, _R), jnp.float32),
        jax.ShapeDtypeStruct((_B, _T, _NRES, _R), jnp.float32),
        jax.ShapeDtypeStruct((_B, _T, _NRES, _R), jnp.float32),
    )
    return pl.pallas_call(
        _tca_kernel_body,
        grid=grid,
        in_specs=[
            pl.BlockSpec((1, 1, _A, _F), lambda b, t: (b, t, 0, 0)),
            pl.BlockSpec((1, _A, _F), lambda b, t: (b, 0, 0)),
            pl.BlockSpec((1, 8, _A), lambda b, t: (b, 0, 0)),
            pl.BlockSpec((_F, _R), lambda b, t: (0, 0)),
            pl.BlockSpec((1, _R), lambda b, t: (0, 0)),
            pl.BlockSpec((3 * _R, _R), lambda b, t: (0, 0)),
        ],
        out_specs=(
            pl.BlockSpec((1, 1, _NRES, _R), lambda b, t: (b, t, 0, 0)),
            pl.BlockSpec((1, 1, _NRES, _R), lambda b, t: (b, t, 0, 0)),
            pl.BlockSpec((1, 1, _NRES, _R), lambda b, t: (b, t, 0, 0)),
        ),
        out_shape=out_shapes,
    )(feats, ressub, reslane, Wa, ba2, Wc)


def _tcb_kernel_body(enc_ref, attr_ref, p1_ref, p2_ref,
                     We_ref, be_ref, Wc_ref, bc_ref, gam_ref, bet_ref,
                     edge_out_ref):
    enc = enc_ref[0, 0, 0]
    attr = attr_ref[0, 0, 0]
    Wc3 = Wc_ref[2 * _R:3 * _R]
    Wep = jnp.dot(We_ref[...], Wc3, preferred_element_type=jnp.float32)
    bprime = bc_ref[...] + jnp.dot(be_ref[...], Wc3,
                                   preferred_element_type=jnp.float32)
    ri = lax.broadcasted_iota(jnp.int32, (_R, 8 * _R), 0)
    ci = lax.broadcasted_iota(jnp.int32, (_R, 8 * _R), 1)
    wtile = jnp.tile(Wep, (8, 8))
    wbig = jnp.where((ri // _DE) == (ci // _R), wtile, 0.0)
    eaw = jnp.dot(attr, wbig, preferred_element_type=jnp.float32)
    eaw = eaw.reshape(_CCELLS // 8, 8, _R).reshape(_CROWS, _NRES, _R)
    i0 = pl.program_id(2) * _CROWS
    P1 = p1_ref[0, 0]
    P2 = p2_ref[0, 0]
    P1c = p1_ref[0, 0, pl.ds(i0, _CROWS), :]
    P2c = p2_ref[0, 0, pl.ds(i0, _CROWS), :]
    A1 = P1c[:, None, :] + P2[None, :, :]
    A2 = P2c[:, None, :] + P1[None, :, :]
    enc3 = lax.broadcast_in_dim(enc, (_CROWS, _NRES, _R), (0, 1))
    phase3 = enc3 < _E
    has3 = enc3 >= 0
    pre = jnp.where(phase3, A1, A2) + eaw + bprime.reshape(1, 1, _R)
    h = jnp.maximum(pre, 0.0)
    mu = jnp.mean(h, axis=-1, keepdims=True)
    var = jnp.mean((h - mu) ** 2, axis=-1, keepdims=True)
    ln = (gam_ref[...].reshape(1, 1, _R) * (h - mu)
          / jnp.sqrt(var + 1e-5) + bet_ref[...].reshape(1, 1, _R))
    edge_out_ref[0, 0] = jnp.where(has3, ln, 0.0)


def _tcb_call(enc5, attr5, p1, p2, We, be2, Wc, bc2, gam2, bet2):
    grid = (_B, _T, _NPART)
    return pl.pallas_call(
        _tcb_kernel_body,
        grid=grid,
        in_specs=[
            pl.BlockSpec((1, 1, 1, _CROWS, _NRES),
                         lambda b, t, ic: (b, t, ic, 0, 0)),
            pl.BlockSpec((1, 1, 1, _CCELLS // 8, _R),
                         lambda b, t, ic: (b, t, ic, 0, 0)),
            pl.BlockSpec((1, 1, _NRES, _R), lambda b, t, ic: (b, t, 0, 0)),
            pl.BlockSpec((1, 1, _NRES, _R), lambda b, t, ic: (b, t, 0, 0)),
            pl.BlockSpec((_DE, _R), lambda b, t, ic: (0, 0)),
            pl.BlockSpec((1, _R), lambda b, t, ic: (0, 0)),
            pl.BlockSpec((3 * _R, _R), lambda b, t, ic: (0, 0)),
            pl.BlockSpec((1, _R), lambda b, t, ic: (0, 0)),
            pl.BlockSpec((1, _R), lambda b, t, ic: (0, 0)),
            pl.BlockSpec((1, _R), lambda b, t, ic: (0, 0)),
        ],
        out_specs=pl.BlockSpec((1, 1, _CROWS, _NRES, _R),
                               lambda b, t, ic: (b, t, ic, 0, 0)),
        out_shape=jax.ShapeDtypeStruct((_B, _T, _NRES, _NRES, _R),
                                       jnp.float32),
    )(enc5, attr5, p1, p2, We, be2, Wc, bc2, gam2, bet2)


def kernel(atom_features, residue_indices, edge_index, edge_attr,
           Wa, ba, We, be, Wc, bc, gamma, beta):
    atom_features = atom_features.astype(jnp.float32)
    residue_indices = residue_indices.astype(jnp.int32)
    edge_index = edge_index.astype(jnp.int32)
    edge_attr = edge_attr.astype(jnp.float32)

    srcdst = edge_index.reshape(-1)
    ca = atom_features[..., 1].reshape(-1)
    resflat = residue_indices.reshape(-1)
    eaflat = edge_attr.reshape(-1, _DE)

    enc_flat, idx_flat = _sc_call(srcdst, ca, resflat)
    easel = _scg_call(idx_flat, eaflat)

    enc5 = enc_flat.reshape(_B, _T, _NPART, _CROWS, _NRES)
    attr5 = easel.reshape(_B, _T, _NPART, _CCELLS // 8, _R)
    ressub = jnp.broadcast_to(residue_indices[:, :, None], (_B, _A, _F))
    reslane = jnp.broadcast_to(residue_indices[:, None, :], (_B, 8, _A))

    node_out, p1, p2 = _tca_call(
        atom_features, ressub, reslane, Wa, ba.reshape(1, _R), Wc)
    edge_out = _tcb_call(
        enc5, attr5, p1, p2, We, be.reshape(1, _R), Wc,
        bc.reshape(1, _R), gamma.reshape(1, _R), beta.reshape(1, _R))
    return (edge_out, node_out)
